# Initial kernel scaffold; baseline (speedup 1.0000x reference)
import jax
import jax.numpy as jnp
from jax.experimental import pallas as pl


def kernel(feats, coords, params):
    raise NotImplementedError("write your pallas kernel here")


# speedup vs baseline: 3.5727x; 3.5727x over previous
import functools

import jax
import jax.numpy as jnp
from jax.experimental import pallas as pl

_IN_CH = 3
_OUT_CH = 512
_MAX_PTS = 2048
_N_PTS = 16384
_VOXEL = 0.005


def _fps_body(npoint, n, x_ref, y_ref, z_ref, o_ref):
    rows = n // 128
    irows = o_ref.shape[0]
    x = x_ref[...]
    y = y_ref[...]
    z = z_ref[...]
    lin = (jax.lax.broadcasted_iota(jnp.int32, (rows, 128), 0) * 128
           + jax.lax.broadcasted_iota(jnp.int32, (rows, 128), 1))
    ilin = (jax.lax.broadcasted_iota(jnp.int32, (irows, 128), 0) * 128
            + jax.lax.broadcasted_iota(jnp.int32, (irows, 128), 1))

    def body(i, carry):
        dists, far, idx = carry
        idx = jnp.where(ilin == i, far, idx)
        mask = lin == far
        cx = jnp.sum(jnp.where(mask, x, 0.0))
        cy = jnp.sum(jnp.where(mask, y, 0.0))
        cz = jnp.sum(jnp.where(mask, z, 0.0))
        dx = x - cx
        dy = y - cy
        dz = z - cz
        d = dx * dx + dy * dy + dz * dz
        dists = jnp.minimum(dists, d)
        m = jnp.max(dists)
        far2 = jnp.min(jnp.where(dists == m, lin, n))
        return dists, far2, idx

    dists0 = jnp.full((rows, 128), 1e10, jnp.float32)
    idx0 = jnp.zeros((irows, 128), jnp.int32)
    _, _, idx = jax.lax.fori_loop(0, npoint, body, (dists0, jnp.int32(0), idx0))
    o_ref[...] = idx


def _fps(xyz, npoint):
    n = xyz.shape[0]
    rows = n // 128
    irows = max(npoint, 128) // 128
    planes = [xyz[:, i].reshape(rows, 128) for i in range(3)]
    out = pl.pallas_call(
        functools.partial(_fps_body, npoint, n),
        out_shape=jax.ShapeDtypeStruct((irows, 128), jnp.int32),
    )(*planes)
    return out.reshape(-1)[:npoint]


def _bn(x, gamma, beta, axes):
    mean = jnp.mean(x, axes, keepdims=True)
    var = jnp.var(x, axes, keepdims=True)
    shp = [1] * x.ndim
    shp[1] = -1
    return gamma.reshape(shp) * (x - mean) / jnp.sqrt(var + 1e-5) + beta.reshape(shp)


def _sqdist(a, b):
    d = (jnp.sum(a * a, -1)[:, :, None] + jnp.sum(b * b, -1)[:, None, :]
         - 2.0 * jnp.einsum('bnd,bmd->bnm', a, b))
    return jnp.maximum(d, 0.0)


def _ball_query(radius, nsample, xyz, new_xyz):
    B, N, _ = xyz.shape
    d = _sqdist(new_xyz, xyz)
    gi = jnp.broadcast_to(jnp.arange(N, dtype=jnp.int32), d.shape)
    gi = jnp.where(d > radius * radius, N, gi)
    gi = jnp.sort(gi, axis=-1)[:, :, :nsample]
    first = gi[:, :, 0:1]
    return jnp.where(gi == N, first, gi)


def _group_gather(points, idx):
    return jnp.take_along_axis(points[:, :, None, :], idx[:, :, :, None], axis=1)


def _sa_module(scales_params, npoint, radii, nsamples, xyz, features):
    fidx = _fps(xyz[0], npoint)[None]
    new_xyz = jnp.take_along_axis(xyz, fidx[:, :, None], axis=1)
    feats_t = jnp.transpose(features, (0, 2, 1))
    outs = []
    for layers, radius, ns in zip(scales_params, radii, nsamples):
        idx = _ball_query(radius, ns, xyz, new_xyz)
        gx = _group_gather(xyz, idx) - new_xyz[:, :, None, :]
        gf = _group_gather(feats_t, idx)
        x = jnp.transpose(jnp.concatenate([gx, gf], axis=-1), (0, 3, 1, 2))
        for L in layers:
            x = jnp.einsum('oc,bcsk->bosk', L["w"], x) + L["b"][None, :, None, None]
            x = _bn(x, L["gamma"], L["beta"], (0, 2, 3))
            x = jax.nn.relu(x)
        outs.append(jnp.max(x, axis=3))
    return new_xyz, jnp.concatenate(outs, axis=1)


def _three_nn(unknown, known):
    d = _sqdist(unknown, known)
    neg, idx = jax.lax.top_k(-d, 3)
    return -neg, idx


def _three_interpolate(feats, idx, weight):
    feats_t = jnp.transpose(feats, (0, 2, 1))
    g = jnp.take_along_axis(feats_t[:, :, None, :], idx[:, :, :, None], axis=1)
    out = jnp.sum(g * weight[:, :, :, None], axis=2)
    return jnp.transpose(out, (0, 2, 1))


def _fp_module(layers, unknown_xyz, known_xyz, unknown_feats, known_feats):
    dist, idx = _three_nn(unknown_xyz, known_xyz)
    dr = 1.0 / (dist + 1e-8)
    w = dr / jnp.sum(dr, -1, keepdims=True)
    interp = _three_interpolate(known_feats, idx, w)
    x = jnp.concatenate([interp, unknown_feats], axis=1)
    for L in layers:
        x = jnp.einsum('oc,bcn->bon', L["w"], x) + L["b"][None, :, None]
        x = _bn(x, L["gamma"], L["beta"], (0, 2))
        x = jax.nn.relu(x)
    return x


def kernel(feats, coords, params):
    xyz = coords[:, 1:4].astype(jnp.float32) * _VOXEL
    xyz_b = xyz[None]
    feats_b = feats[None]
    fps_idx = _fps(xyz, _MAX_PTS)[None]
    xyz_sub = jnp.take_along_axis(xyz_b, fps_idx[:, :, None], axis=1)
    feats_sub = jnp.take_along_axis(feats_b, fps_idx[:, :, None], axis=1)
    f0 = jnp.transpose(feats_sub, (0, 2, 1))
    l1x, l1f = _sa_module(params["sa1"], 512, [0.02, 0.04], [16, 16], xyz_sub, f0)
    l2x, l2f = _sa_module(params["sa2"], 128, [0.04, 0.08], [16, 16], l1x, l1f)
    l3x, l3f = _sa_module(params["sa3"], 32, [0.08, 0.16], [16, 16], l2x, l2f)
    l2f = _fp_module(params["fp3"], l2x, l3x, l2f, l3f)
    l1f = _fp_module(params["fp2"], l1x, l2x, l1f, l2f)
    l0f = _fp_module(params["fp1"], xyz_sub, l1x, f0, l1f)
    dist, idx = _three_nn(xyz_b, xyz_sub)
    dr = 1.0 / (dist + 1e-8)
    w = dr / jnp.sum(dr, -1, keepdims=True)
    l0f = _three_interpolate(l0f, idx, w)
    fin = params["final"]
    out = jnp.einsum('oc,bcn->bon', fin["w"], l0f) + fin["b"][None, :, None]
    out = _bn(out, fin["gamma"], fin["beta"], (0, 2))
    return jnp.transpose(out, (0, 2, 1))[0]


# speedup vs baseline: 13.1185x; 3.6719x over previous
import functools

import jax
import jax.numpy as jnp
from jax.experimental import pallas as pl

_IN_CH = 3
_OUT_CH = 512
_MAX_PTS = 2048
_N_PTS = 16384
_VOXEL = 0.005


def _fps_body(npoint, n, x_ref, y_ref, z_ref, o_ref):
    rows = n // 128
    irows = o_ref.shape[0]
    x = x_ref[...]
    y = y_ref[...]
    z = z_ref[...]
    lin = (jax.lax.broadcasted_iota(jnp.int32, (rows, 128), 0) * 128
           + jax.lax.broadcasted_iota(jnp.int32, (rows, 128), 1))
    ilin = (jax.lax.broadcasted_iota(jnp.int32, (irows, 128), 0) * 128
            + jax.lax.broadcasted_iota(jnp.int32, (irows, 128), 1))

    def body(i, carry):
        dists, far, idx = carry
        idx = jnp.where(ilin == i, far, idx)
        mask = lin == far
        cx = jnp.sum(jnp.where(mask, x, 0.0))
        cy = jnp.sum(jnp.where(mask, y, 0.0))
        cz = jnp.sum(jnp.where(mask, z, 0.0))
        dx = x - cx
        dy = y - cy
        dz = z - cz
        d = dx * dx + dy * dy + dz * dz
        dists = jnp.minimum(dists, d)
        m = jnp.max(dists)
        far2 = jnp.min(jnp.where(dists == m, lin, n))
        return dists, far2, idx

    dists0 = jnp.full((rows, 128), 1e10, jnp.float32)
    idx0 = jnp.zeros((irows, 128), jnp.int32)
    _, _, idx = jax.lax.fori_loop(0, npoint, body, (dists0, jnp.int32(0), idx0))
    o_ref[...] = idx


def _fps(xyz, npoint):
    n = xyz.shape[0]
    rows = n // 128
    irows = max(npoint, 128) // 128
    planes = [xyz[:, i].reshape(rows, 128) for i in range(3)]
    out = pl.pallas_call(
        functools.partial(_fps_body, npoint, n),
        out_shape=jax.ShapeDtypeStruct((irows, 128), jnp.int32),
    )(*planes)
    return out.reshape(-1)[:npoint]


_RB = 512


def _final_body(d_ref, f_ref, w_ref, o_ref, st_ref):
    npnt = d_ref.shape[1]
    d = d_ref[...]
    colio = jax.lax.broadcasted_iota(jnp.int32, (_RB, npnt), 1)

    m1 = jnp.min(d, axis=1, keepdims=True)
    i1 = jnp.min(jnp.where(d == m1, colio, npnt), axis=1, keepdims=True)
    d1 = jnp.where(colio == i1, 1e30, d)
    m2 = jnp.min(d1, axis=1, keepdims=True)
    i2 = jnp.min(jnp.where(d1 == m2, colio, npnt), axis=1, keepdims=True)
    d2 = jnp.where(colio == i2, 1e30, d1)
    m3 = jnp.min(d2, axis=1, keepdims=True)
    i3 = jnp.min(jnp.where(d2 == m3, colio, npnt), axis=1, keepdims=True)

    r1 = 1.0 / (m1 + 1e-8)
    r2 = 1.0 / (m2 + 1e-8)
    r3 = 1.0 / (m3 + 1e-8)
    rs = (r1 + r2) + r3
    s = (jnp.where(colio == i1, r1 / rs, 0.0)
         + jnp.where(colio == i2, r2 / rs, 0.0)
         + jnp.where(colio == i3, r3 / rs, 0.0))
    interp = jnp.dot(s, f_ref[...], preferred_element_type=jnp.float32)
    y = jnp.dot(interp, w_ref[...], preferred_element_type=jnp.float32)
    o_ref[...] = y
    st_ref[0, 0:1, :] = jnp.sum(y, axis=0, keepdims=True)
    st_ref[0, 1:2, :] = jnp.sum(y * y, axis=0, keepdims=True)


def _final_stage(xyz, xyz_sub, f_t, fin):
    n = xyz.shape[0]
    m = xyz_sub.shape[0]
    nblk = n // _RB
    cout = fin["w"].shape[0]
    d = _sqdist(xyz[None], xyz_sub[None])[0]
    y, stats = pl.pallas_call(
        _final_body,
        grid=(nblk,),
        in_specs=[
            pl.BlockSpec((_RB, m), lambda i: (i, 0)),
            pl.BlockSpec((m, f_t.shape[0]), lambda i: (0, 0)),
            pl.BlockSpec((f_t.shape[0], cout), lambda i: (0, 0)),
        ],
        out_specs=[
            pl.BlockSpec((_RB, cout), lambda i: (i, 0)),
            pl.BlockSpec((1, 2, cout), lambda i: (i, 0, 0)),
        ],
        out_shape=[
            jax.ShapeDtypeStruct((n, cout), jnp.float32),
            jax.ShapeDtypeStruct((nblk, 2, cout), jnp.float32),
        ],
    )(d, f_t.T, fin["w"].T)
    s_tot = jnp.sum(stats[:, 0, :], axis=0)
    ss_tot = jnp.sum(stats[:, 1, :], axis=0)
    mean = s_tot / n
    var = jnp.maximum(ss_tot / n - mean * mean, 0.0)
    rinv = fin["gamma"] / jnp.sqrt(var + 1e-5)
    return rinv[None, :] * (y - mean[None, :]) + fin["beta"][None, :]


def _bn(x, gamma, beta, axes):
    mean = jnp.mean(x, axes, keepdims=True)
    var = jnp.var(x, axes, keepdims=True)
    shp = [1] * x.ndim
    shp[1] = -1
    return gamma.reshape(shp) * (x - mean) / jnp.sqrt(var + 1e-5) + beta.reshape(shp)


def _sqdist(a, b):
    d = (jnp.sum(a * a, -1)[:, :, None] + jnp.sum(b * b, -1)[:, None, :]
         - 2.0 * jnp.einsum('bnd,bmd->bnm', a, b))
    return jnp.maximum(d, 0.0)


def _ball_query(radius, nsample, xyz, new_xyz):
    B, N, _ = xyz.shape
    d = _sqdist(new_xyz, xyz)
    gi = jnp.broadcast_to(jnp.arange(N, dtype=jnp.int32), d.shape)
    gi = jnp.where(d > radius * radius, N, gi)
    gi = jnp.sort(gi, axis=-1)[:, :, :nsample]
    first = gi[:, :, 0:1]
    return jnp.where(gi == N, first, gi)


def _group_gather(points, idx):
    return jnp.take_along_axis(points[:, :, None, :], idx[:, :, :, None], axis=1)


def _sa_module(scales_params, npoint, radii, nsamples, xyz, features):
    fidx = _fps(xyz[0], npoint)[None]
    new_xyz = jnp.take_along_axis(xyz, fidx[:, :, None], axis=1)
    feats_t = jnp.transpose(features, (0, 2, 1))
    outs = []
    for layers, radius, ns in zip(scales_params, radii, nsamples):
        idx = _ball_query(radius, ns, xyz, new_xyz)
        gx = _group_gather(xyz, idx) - new_xyz[:, :, None, :]
        gf = _group_gather(feats_t, idx)
        x = jnp.transpose(jnp.concatenate([gx, gf], axis=-1), (0, 3, 1, 2))
        for L in layers:
            x = jnp.einsum('oc,bcsk->bosk', L["w"], x) + L["b"][None, :, None, None]
            x = _bn(x, L["gamma"], L["beta"], (0, 2, 3))
            x = jax.nn.relu(x)
        outs.append(jnp.max(x, axis=3))
    return new_xyz, jnp.concatenate(outs, axis=1)


def _three_nn(unknown, known):
    d = _sqdist(unknown, known)
    neg, idx = jax.lax.top_k(-d, 3)
    return -neg, idx


def _three_interpolate(feats, idx, weight):
    feats_t = jnp.transpose(feats, (0, 2, 1))
    g = jnp.take_along_axis(feats_t[:, :, None, :], idx[:, :, :, None], axis=1)
    out = jnp.sum(g * weight[:, :, :, None], axis=2)
    return jnp.transpose(out, (0, 2, 1))


def _fp_module(layers, unknown_xyz, known_xyz, unknown_feats, known_feats):
    dist, idx = _three_nn(unknown_xyz, known_xyz)
    dr = 1.0 / (dist + 1e-8)
    w = dr / jnp.sum(dr, -1, keepdims=True)
    interp = _three_interpolate(known_feats, idx, w)
    x = jnp.concatenate([interp, unknown_feats], axis=1)
    for L in layers:
        x = jnp.einsum('oc,bcn->bon', L["w"], x) + L["b"][None, :, None]
        x = _bn(x, L["gamma"], L["beta"], (0, 2))
        x = jax.nn.relu(x)
    return x


def kernel(feats, coords, params):
    xyz = coords[:, 1:4].astype(jnp.float32) * _VOXEL
    xyz_b = xyz[None]
    feats_b = feats[None]
    fps_idx = _fps(xyz, _MAX_PTS)[None]
    xyz_sub = jnp.take_along_axis(xyz_b, fps_idx[:, :, None], axis=1)
    feats_sub = jnp.take_along_axis(feats_b, fps_idx[:, :, None], axis=1)
    f0 = jnp.transpose(feats_sub, (0, 2, 1))
    l1x, l1f = _sa_module(params["sa1"], 512, [0.02, 0.04], [16, 16], xyz_sub, f0)
    l2x, l2f = _sa_module(params["sa2"], 128, [0.04, 0.08], [16, 16], l1x, l1f)
    l3x, l3f = _sa_module(params["sa3"], 32, [0.08, 0.16], [16, 16], l2x, l2f)
    l2f = _fp_module(params["fp3"], l2x, l3x, l2f, l3f)
    l1f = _fp_module(params["fp2"], l1x, l2x, l1f, l2f)
    l0f = _fp_module(params["fp1"], xyz_sub, l1x, f0, l1f)
    return _final_stage(xyz, xyz_sub[0], l0f[0], params["final"])


# speedup vs baseline: 18.4031x; 1.4028x over previous
import functools

import jax
import jax.numpy as jnp
from jax.experimental import pallas as pl

_IN_CH = 3
_OUT_CH = 512
_MAX_PTS = 2048
_N_PTS = 16384
_VOXEL = 0.005


def _fps_body(npoint, n, x_ref, y_ref, z_ref, o_ref):
    rows = n // 128
    irows = o_ref.shape[0]
    x = x_ref[...]
    y = y_ref[...]
    z = z_ref[...]
    lin = (jax.lax.broadcasted_iota(jnp.int32, (rows, 128), 0) * 128
           + jax.lax.broadcasted_iota(jnp.int32, (rows, 128), 1))
    ilin = (jax.lax.broadcasted_iota(jnp.int32, (irows, 128), 0) * 128
            + jax.lax.broadcasted_iota(jnp.int32, (irows, 128), 1))

    lane = jax.lax.broadcasted_iota(jnp.int32, (1, 128), 1)

    def body(i, carry):
        dists, far, idx = carry
        idx = jnp.where(ilin == i, far, idx)
        fr = far // 128
        fc = far - fr * 128
        lm = lane == fc
        cx = jnp.sum(jnp.where(lm, x_ref[pl.ds(fr, 1), :], 0.0))
        cy = jnp.sum(jnp.where(lm, y_ref[pl.ds(fr, 1), :], 0.0))
        cz = jnp.sum(jnp.where(lm, z_ref[pl.ds(fr, 1), :], 0.0))
        dx = x - cx
        dy = y - cy
        dz = z - cz
        d = dx * dx + dy * dy + dz * dz
        dists = jnp.minimum(dists, d)
        m = jnp.max(dists)
        far2 = jnp.min(jnp.where(dists == m, lin, n))
        return dists, far2, idx

    dists0 = jnp.full((rows, 128), 1e10, jnp.float32)
    idx0 = jnp.zeros((irows, 128), jnp.int32)
    _, _, idx = jax.lax.fori_loop(0, npoint, body, (dists0, jnp.int32(0), idx0))
    o_ref[...] = idx


def _fps(xyz, npoint):
    n = xyz.shape[0]
    rows = n // 128
    irows = max(npoint, 128) // 128
    planes = [xyz[:, i].reshape(rows, 128) for i in range(3)]
    out = pl.pallas_call(
        functools.partial(_fps_body, npoint, n),
        out_shape=jax.ShapeDtypeStruct((irows, 128), jnp.int32),
    )(*planes)
    return out.reshape(-1)[:npoint]


_RB = 512


def _final_body(d_ref, f_ref, w_ref, o_ref, st_ref):
    npnt = d_ref.shape[1]
    d = d_ref[...]
    colio = jax.lax.broadcasted_iota(jnp.int32, (_RB, npnt), 1)

    m1 = jnp.min(d, axis=1, keepdims=True)
    i1 = jnp.min(jnp.where(d == m1, colio, npnt), axis=1, keepdims=True)
    d1 = jnp.where(colio == i1, 1e30, d)
    m2 = jnp.min(d1, axis=1, keepdims=True)
    i2 = jnp.min(jnp.where(d1 == m2, colio, npnt), axis=1, keepdims=True)
    d2 = jnp.where(colio == i2, 1e30, d1)
    m3 = jnp.min(d2, axis=1, keepdims=True)
    i3 = jnp.min(jnp.where(d2 == m3, colio, npnt), axis=1, keepdims=True)

    r1 = 1.0 / (m1 + 1e-8)
    r2 = 1.0 / (m2 + 1e-8)
    r3 = 1.0 / (m3 + 1e-8)
    rs = (r1 + r2) + r3
    s = (jnp.where(colio == i1, r1 / rs, 0.0)
         + jnp.where(colio == i2, r2 / rs, 0.0)
         + jnp.where(colio == i3, r3 / rs, 0.0))
    interp = jnp.dot(s, f_ref[...], preferred_element_type=jnp.float32)
    y = jnp.dot(interp, w_ref[...], preferred_element_type=jnp.float32)
    o_ref[...] = y
    st_ref[0, 0:1, :] = jnp.sum(y, axis=0, keepdims=True)
    st_ref[0, 1:2, :] = jnp.sum(y * y, axis=0, keepdims=True)


def _final_stage(xyz, xyz_sub, f, fin):
    n = xyz.shape[0]
    m = xyz_sub.shape[0]
    nblk = n // _RB
    cout = fin["w"].shape[0]
    d = _sqdist(xyz[None], xyz_sub[None])[0]
    y, stats = pl.pallas_call(
        _final_body,
        grid=(nblk,),
        in_specs=[
            pl.BlockSpec((_RB, m), lambda i: (i, 0)),
            pl.BlockSpec((m, f.shape[1]), lambda i: (0, 0)),
            pl.BlockSpec((f.shape[1], cout), lambda i: (0, 0)),
        ],
        out_specs=[
            pl.BlockSpec((_RB, cout), lambda i: (i, 0)),
            pl.BlockSpec((1, 2, cout), lambda i: (i, 0, 0)),
        ],
        out_shape=[
            jax.ShapeDtypeStruct((n, cout), jnp.float32),
            jax.ShapeDtypeStruct((nblk, 2, cout), jnp.float32),
        ],
    )(d, f, fin["w"].T)
    s_tot = jnp.sum(stats[:, 0, :], axis=0)
    ss_tot = jnp.sum(stats[:, 1, :], axis=0)
    mean = s_tot / n
    var = jnp.maximum(ss_tot / n - mean * mean, 0.0)
    rinv = fin["gamma"] / jnp.sqrt(var + 1e-5)
    return rinv[None, :] * (y - mean[None, :]) + fin["beta"][None, :]


def _sa_body(r2, nsamp, d_ref, xf_ref, cq_ref, w1_ref, p1_ref, w2_ref, p2_ref,
             o_ref):
    q, p = d_ref.shape
    d = d_ref[...]
    col = jax.lax.broadcasted_iota(jnp.int32, (q, p), 1)
    avail = jnp.where(d > r2, p, col)
    i0 = jnp.min(avail, axis=1, keepdims=True)
    iks = []
    cur = avail
    for _ in range(nsamp):
        ik = jnp.min(cur, axis=1, keepdims=True)
        iks.append(ik)
        cur = jnp.where(col == ik, p, cur)
    iks = [jnp.where(ik == p, i0, ik) for ik in iks]

    h_all = jnp.dot(xf_ref[...], w1_ref[...], preferred_element_type=jnp.float32)
    cq = jnp.dot(cq_ref[...], w1_ref[0:3, :], preferred_element_type=jnp.float32)
    hs = []
    for ik in iks:
        g = (col == ik).astype(jnp.float32)
        hs.append(jnp.dot(g, h_all, preferred_element_type=jnp.float32) - cq)
    ns = q * nsamp
    s1 = sum([jnp.sum(h, axis=0, keepdims=True) for h in hs])
    ss1 = sum([jnp.sum(h * h, axis=0, keepdims=True) for h in hs])
    mu1 = s1 / ns
    v1 = jnp.maximum(ss1 / ns - mu1 * mu1, 0.0)
    sc1 = p1_ref[0:1, :] / jnp.sqrt(v1 + 1e-5)
    b1 = p1_ref[1:2, :]
    ts = [jnp.dot(jax.nn.relu(sc1 * (h - mu1) + b1), w2_ref[...],
                  preferred_element_type=jnp.float32) for h in hs]
    s2 = sum([jnp.sum(t, axis=0, keepdims=True) for t in ts])
    ss2 = sum([jnp.sum(t * t, axis=0, keepdims=True) for t in ts])
    mu2 = s2 / ns
    v2 = jnp.maximum(ss2 / ns - mu2 * mu2, 0.0)
    sc2 = p2_ref[0:1, :] / jnp.sqrt(v2 + 1e-5)
    b2 = p2_ref[1:2, :]
    rs = [jax.nn.relu(sc2 * (t - mu2) + b2) for t in ts]
    acc = rs[0]
    for r in rs[1:]:
        acc = jnp.maximum(acc, r)
    o_ref[...] = acc


def _sa_scale(d, xf, cq, layers, r2, nsamp):
    q = d.shape[0]
    l1, l2 = layers
    c2 = l2["w"].shape[0]
    p1 = jnp.stack([l1["gamma"], l1["beta"]])
    p2 = jnp.stack([l2["gamma"], l2["beta"]])
    return pl.pallas_call(
        functools.partial(_sa_body, r2, nsamp),
        out_shape=jax.ShapeDtypeStruct((q, c2), jnp.float32),
    )(d, xf, cq, l1["w"].T, p1, l2["w"].T, p2)


def _sa_module_p(scales_params, npoint, radii, nsamples, xyz, feats_r):
    fidx = _fps(xyz[0], npoint)[None]
    new_xyz = jnp.take_along_axis(xyz, fidx[:, :, None], axis=1)
    d = _sqdist(new_xyz, xyz)[0]
    xf = jnp.concatenate([xyz[0], feats_r], axis=1)
    outs = []
    for layers, radius, ns in zip(scales_params, radii, nsamples):
        r2 = float(radius * radius)
        outs.append(_sa_scale(d, xf, new_xyz[0], layers, r2, ns))
    return new_xyz, jnp.concatenate(outs, axis=1)


def _fp_body(ck, d_ref, kf_ref, sf_ref, w1_ref, p1_ref, w2_ref, p2_ref, o_ref):
    q, m = d_ref.shape
    d = d_ref[...]
    colio = jax.lax.broadcasted_iota(jnp.int32, (q, m), 1)
    m1 = jnp.min(d, axis=1, keepdims=True)
    i1 = jnp.min(jnp.where(d == m1, colio, m), axis=1, keepdims=True)
    d1 = jnp.where(colio == i1, 1e30, d)
    m2 = jnp.min(d1, axis=1, keepdims=True)
    i2 = jnp.min(jnp.where(d1 == m2, colio, m), axis=1, keepdims=True)
    d2 = jnp.where(colio == i2, 1e30, d1)
    m3 = jnp.min(d2, axis=1, keepdims=True)
    i3 = jnp.min(jnp.where(d2 == m3, colio, m), axis=1, keepdims=True)
    r1 = 1.0 / (m1 + 1e-8)
    r2 = 1.0 / (m2 + 1e-8)
    r3 = 1.0 / (m3 + 1e-8)
    rs = (r1 + r2) + r3
    s = (jnp.where(colio == i1, r1 / rs, 0.0)
         + jnp.where(colio == i2, r2 / rs, 0.0)
         + jnp.where(colio == i3, r3 / rs, 0.0))
    interp = jnp.dot(s, kf_ref[...], preferred_element_type=jnp.float32)
    t1 = (jnp.dot(interp, w1_ref[0:ck, :], preferred_element_type=jnp.float32)
          + jnp.dot(sf_ref[...], w1_ref[ck:, :], preferred_element_type=jnp.float32))
    mu1 = jnp.mean(t1, axis=0, keepdims=True)
    v1 = jnp.maximum(jnp.mean(t1 * t1, axis=0, keepdims=True) - mu1 * mu1, 0.0)
    z1 = jax.nn.relu(p1_ref[0:1, :] / jnp.sqrt(v1 + 1e-5) * (t1 - mu1)
                     + p1_ref[1:2, :])
    t2 = jnp.dot(z1, w2_ref[...], preferred_element_type=jnp.float32)
    mu2 = jnp.mean(t2, axis=0, keepdims=True)
    v2 = jnp.maximum(jnp.mean(t2 * t2, axis=0, keepdims=True) - mu2 * mu2, 0.0)
    o_ref[...] = jax.nn.relu(p2_ref[0:1, :] / jnp.sqrt(v2 + 1e-5) * (t2 - mu2)
                             + p2_ref[1:2, :])


def _fp_module_p(layers, unknown_xyz, known_xyz, sf, kf):
    d = _sqdist(unknown_xyz, known_xyz)[0]
    l1, l2 = layers
    ck = kf.shape[1]
    c2 = l2["w"].shape[0]
    p1 = jnp.stack([l1["gamma"], l1["beta"]])
    p2 = jnp.stack([l2["gamma"], l2["beta"]])
    return pl.pallas_call(
        functools.partial(_fp_body, ck),
        out_shape=jax.ShapeDtypeStruct((d.shape[0], c2), jnp.float32),
    )(d, kf, sf, l1["w"].T, p1, l2["w"].T, p2)


def _sqdist(a, b):
    d = (jnp.sum(a * a, -1)[:, :, None] + jnp.sum(b * b, -1)[:, None, :]
         - 2.0 * jnp.einsum('bnd,bmd->bnm', a, b))
    return jnp.maximum(d, 0.0)


def kernel(feats, coords, params):
    xyz = coords[:, 1:4].astype(jnp.float32) * _VOXEL
    xyz_b = xyz[None]
    feats_b = feats[None]
    fps_idx = _fps(xyz, _MAX_PTS)[None]
    xyz_sub = jnp.take_along_axis(xyz_b, fps_idx[:, :, None], axis=1)
    f0_r = jnp.take_along_axis(feats_b, fps_idx[:, :, None], axis=1)[0]
    l1x, l1f_r = _sa_module_p(params["sa1"], 512, [0.02, 0.04], [16, 16],
                              xyz_sub, f0_r)
    l2x, l2f_r = _sa_module_p(params["sa2"], 128, [0.04, 0.08], [16, 16],
                              l1x, l1f_r)
    l3x, l3f_r = _sa_module_p(params["sa3"], 32, [0.08, 0.16], [16, 16],
                              l2x, l2f_r)
    l2f_r = _fp_module_p(params["fp3"], l2x, l3x, l2f_r, l3f_r)
    l1f_r = _fp_module_p(params["fp2"], l1x, l2x, l1f_r, l2f_r)
    l0f_r = _fp_module_p(params["fp1"], xyz_sub, l1x, f0_r, l1f_r)
    return _final_stage(xyz, xyz_sub[0], l0f_r, params["final"])
